# baseline (speedup 1.0000x reference)
import jax
import jax.numpy as jnp
from jax import lax
from jax.experimental import pallas as pl
from jax.experimental.pallas import tpu as pltpu

_B, _P, _NOBJ, _NC = 16, 16320, 12, 21
_R, _C = 8, 2048
_PF = _R * _C
_W = 128
_NCH = _C // _W
_THRESHOLD, _NEG_POS_RATIO, _THETA, _ALPHA = 0.5, 3, 0.01, 1.0
_NROW = 2 * _B


def _chunk_idx(c):
    r = lax.broadcasted_iota(jnp.int32, (_R, _W), 0)
    l = lax.broadcasted_iota(jnp.int32, (_R, _W), 1)
    return r * _C + (c * _W + l)


def _full_idx():
    r = lax.broadcasted_iota(jnp.int32, (_R, _C), 0)
    l = lax.broadcasted_iota(jnp.int32, (_R, _C), 1)
    return r * _C + l


def _box_scalars(boxes_ref):
    out = []
    for j in range(_NOBJ):
        bx1 = boxes_ref[0, j, 0]
        by1 = boxes_ref[0, j, 1]
        bx2 = boxes_ref[0, j, 2]
        by2 = boxes_ref[0, j, 3]
        out.append((bx1, by1, bx2, by2, (bx2 - bx1) * (by2 - by1)))
    return out

def _iou_stage_a(bx, anchors_fn, iou_scr, best_scr, obj_scr):
    for c in range(_NCH):
        sl = pl.ds(c * _W, _W)
        ax1, ay1, ax2, ay2 = anchors_fn(c)
        area_b = (ax2 - ax1) * (ay2 - ay1)
        best = None
        obj = None
        for j in range(_NOBJ):
            bx1, by1, bx2, by2, area_a = bx[j]
            w = jnp.maximum(jnp.minimum(bx2, ax2) - jnp.maximum(bx1, ax1),
                            0.0)
            h = jnp.maximum(jnp.minimum(by2, ay2) - jnp.maximum(by1, ay1),
                            0.0)
            inter = w * h
            iou = inter / (area_a + area_b - inter)
            iou_scr[j, :, sl] = iou
            if j == 0:
                best = iou
                obj = jnp.zeros((_R, _W), jnp.int32)
            else:
                obj = jnp.where(iou > best, j, obj)
                best = jnp.maximum(best, iou)
        best_scr[:, sl] = best
        obj_scr[:, sl] = obj


def _rank_stage_b(iou_scr):
    fidx = _full_idx()
    pfs, svs = [], []
    rank = jnp.zeros((1, 1), jnp.int32) - 1
    for j in range(_NOBJ):
        ioj = iou_scr[j]
        mx = jnp.max(ioj, axis=(0, 1), keepdims=True)
        mxb = jnp.broadcast_to(mx, (_R, _C))
        pf = jnp.min(jnp.where(ioj == mxb, fidx, _PF), axis=(0, 1),
                     keepdims=True)
        valid = mx > 0.0
        rank = rank + valid.astype(jnp.int32)
        sv = jnp.where(valid, rank + 1, 0)
        svs.append(jnp.broadcast_to(sv, (_R, _W)))
        pfs.append(jnp.broadcast_to(pf, (_R, _W)))
    return pfs, svs


def _finish_chunk(c, bx, lbl, best_scr, obj_scr, pfs, svs, cxcy):
    sl = pl.ds(c * _W, _W)
    fidx = _chunk_idx(c)
    best = best_scr[:, sl]
    obj = obj_scr[:, sl]
    force = jnp.zeros((_R, _W), jnp.int32)
    for j in range(_NOBJ):
        force = jnp.maximum(force, jnp.where(fidx == pfs[j], svs[j], 0))
    forced = force > 0
    best = jnp.where(forced, 1.0, best)
    obj = jnp.where(forced, force - 1, obj)
    lab = jnp.zeros((_R, _W), jnp.int32)
    gx1 = jnp.zeros((_R, _W), jnp.float32)
    gy1 = jnp.zeros((_R, _W), jnp.float32)
    gx2 = jnp.zeros((_R, _W), jnp.float32)
    gy2 = jnp.zeros((_R, _W), jnp.float32)
    for j in range(_NOBJ):
        sel = obj == j
        lab = jnp.where(sel, lbl[j], lab)
        gx1 = jnp.where(sel, bx[j][0], gx1)
        gy1 = jnp.where(sel, bx[j][1], gy1)
        gx2 = jnp.where(sel, bx[j][2], gx2)
        gy2 = jnp.where(sel, bx[j][3], gy2)
    lab = jnp.where(best < _THRESHOLD, 0, lab)
    pcx, pcy, pw, ph = cxcy
    cx = (gx1 + gx2) / 2.0
    cy = (gy1 + gy2) / 2.0
    w = gx2 - gx1
    h = gy2 - gy1
    t0 = (cx - pcx) / (pw / 10.0)
    t1 = (cy - pcy) / (ph / 10.0)
    t2 = jnp.log(w / pw) * 5.0
    t3 = jnp.log(h / ph) * 5.0
    return lab, (t0, t1, t2, t3)


def _pad_mask(c, x):
    if c == _NCH - 1:
        return jnp.where(_chunk_idx(c) < _P, x, 0.0)
    return x


def _batched_topk_sums(neg_ref, kv_ref):
    kcol = kv_ref[:, 0:1]

    def count_ge(mid):
        cnt = jnp.zeros((_NROW, 128), jnp.int32)
        m3 = mid[:, None, :]
        for c in range(_NCH):
            blk = lax.bitcast_convert_type(
                neg_ref[:, :, pl.ds(c * _W, _W)], jnp.int32)
            cnt = cnt + jnp.sum((blk >= m3).astype(jnp.int32), axis=1)
        return jnp.sum(cnt, axis=1, keepdims=True)

    def body(_, lohi):
        lo, hi = lohi
        mid = lo + (hi - lo + 1) // 2
        ok = count_ge(mid) >= kcol
        return jnp.where(ok, mid, lo), jnp.where(ok, hi, mid - 1)

    lo, _ = lax.fori_loop(
        0, 31, body,
        (jnp.zeros((_NROW, 1), jnp.int32),
         jnp.full((_NROW, 1), 0x7F800000, jnp.int32)))
    tf = lax.bitcast_convert_type(lo, jnp.float32)
    srel = jnp.zeros((_NROW, 128), jnp.float32)
    t3 = tf[:, None, :]
    for c in range(_NCH):
        blk = neg_ref[:, :, pl.ds(c * _W, _W)]
        srel = srel + jnp.sum(jnp.maximum(blk - t3, 0.0), axis=1)
    s = jnp.sum(srel, axis=1, keepdims=True)
    kf = kcol.astype(jnp.float32)
    hard = jnp.where(kcol > 0, kf * tf + s, 0.0)
    return jnp.sum(hard[:_B, 0]), jnp.sum(hard[_B:, 0])


def _body(pr_ref, boxes_ref, labels_ref, al_ref, as_ref, ol_ref, os_ref,
          out_ref, acc_ref, neg_ref, kv_ref, iou_scr, best_scr, obj_scr,
          dec_scr):
    i = pl.program_id(0)
    bx = _box_scalars(boxes_ref)
    lbl = [labels_ref[0, 0, j] for j in range(_NOBJ)]

    def arm_anchors(c):
        sl = pl.ds(c * _W, _W)
        pcx, pcy, pw, ph = (pr_ref[0, :, sl], pr_ref[1, :, sl],
                            pr_ref[2, :, sl], pr_ref[3, :, sl])
        return (pcx - pw / 2.0, pcy - ph / 2.0,
                pcx + pw / 2.0, pcy + ph / 2.0)

    _iou_stage_a(bx, arm_anchors, iou_scr, best_scr, obj_scr)
    pfs, svs = _rank_stage_b(iou_scr)
    loc_ps = jnp.zeros((_R, _W), jnp.float32)
    cpos_ps = jnp.zeros((_R, _W), jnp.float32)
    np_ps = jnp.zeros((_R, _W), jnp.int32)
    for c in range(_NCH):
        sl = pl.ds(c * _W, _W)
        cxcy = (pr_ref[0, :, sl], pr_ref[1, :, sl],
                pr_ref[2, :, sl], pr_ref[3, :, sl])
        lab, ta = _finish_chunk(c, bx, lbl, best_scr, obj_scr, pfs, svs,
                                cxcy)
        pos = lab > 0
        posf = pos.astype(jnp.float32)
        for k in range(4):
            d = jnp.abs(al_ref[0, k, :, sl] - ta[k])
            loc_ps = loc_ps + jnp.where(d < 1.0, 0.5 * d * d,
                                        d - 0.5) * posf
        s0 = as_ref[0, 0, :, sl]
        s1 = as_ref[0, 1, :, sl]
        m2 = jnp.maximum(s0, s1)
        lse = m2 + jnp.log(jnp.exp(s0 - m2) + jnp.exp(s1 - m2))
        ce = lse - jnp.where(pos, s1, s0)
        cpos_ps = cpos_ps + ce * posf
        np_ps = np_ps + pos.astype(jnp.int32)
        neg_ref[i, :, sl] = _pad_mask(c, jnp.where(pos, 0.0, ce))
    n_pos_a = jnp.sum(np_ps)
    kv_ref[pl.ds(i, 1), :] = jnp.full((1, 128), _NEG_POS_RATIO * n_pos_a,
                                      jnp.int32)
    loc_a = jnp.sum(loc_ps)
    cpos_a = jnp.sum(cpos_ps)

    def odm_anchors(c):
        sl = pl.ds(c * _W, _W)
        pcx, pcy, pw, ph = (pr_ref[0, :, sl], pr_ref[1, :, sl],
                            pr_ref[2, :, sl], pr_ref[3, :, sl])
        a0, a1, a2, a3 = (al_ref[0, 0, :, sl], al_ref[0, 1, :, sl],
                          al_ref[0, 2, :, sl], al_ref[0, 3, :, sl])
        dcx = a0 * pw / 10.0 + pcx
        dcy = a1 * ph / 10.0 + pcy
        dw = jnp.exp(a2 / 5.0) * pw
        dh = jnp.exp(a3 / 5.0) * ph
        dx1 = dcx - dw / 2.0
        dy1 = dcy - dh / 2.0
        dx2 = dcx + dw / 2.0
        dy2 = dcy + dh / 2.0
        dec_scr[0, :, sl] = (dx1 + dx2) / 2.0
        dec_scr[1, :, sl] = (dy1 + dy2) / 2.0
        dec_scr[2, :, sl] = dx2 - dx1
        dec_scr[3, :, sl] = dy2 - dy1
        return dx1, dy1, dx2, dy2

    _iou_stage_a(bx, odm_anchors, iou_scr, best_scr, obj_scr)
    pfs, svs = _rank_stage_b(iou_scr)
    loc_ps = jnp.zeros((_R, _W), jnp.float32)
    cpos_ps = jnp.zeros((_R, _W), jnp.float32)
    np_ps = jnp.zeros((_R, _W), jnp.int32)
    for c in range(_NCH):
        sl = pl.ds(c * _W, _W)
        cxcy = (dec_scr[0, :, sl], dec_scr[1, :, sl],
                dec_scr[2, :, sl], dec_scr[3, :, sl])
        lab, to = _finish_chunk(c, bx, lbl, best_scr, obj_scr, pfs, svs,
                                cxcy)
        s0 = as_ref[0, 0, :, sl]
        s1 = as_ref[0, 1, :, sl]
        em = jnp.maximum(s0, s1)
        e0 = jnp.exp(s0 - em)
        e1 = jnp.exp(s1 - em)
        easy = e1 / (e0 + e1) < _THETA
        pos = jnp.logical_and(lab > 0, jnp.logical_not(easy))
        posf = pos.astype(jnp.float32)
        for k in range(4):
            d = jnp.abs(ol_ref[0, k, :, sl] - to[k])
            loc_ps = loc_ps + jnp.where(d < 1.0, 0.5 * d * d,
                                        d - 0.5) * posf
        sc = [os_ref[0, cc, :, sl] for cc in range(_NC)]
        mo = sc[0]
        for cc in range(1, _NC):
            mo = jnp.maximum(mo, sc[cc])
        se = jnp.zeros((_R, _W), jnp.float32)
        st = jnp.zeros((_R, _W), jnp.float32)
        for cc in range(_NC):
            se = se + jnp.exp(sc[cc] - mo)
            st = st + jnp.where(lab == cc, sc[cc], 0.0)
        ce = (mo + jnp.log(se)) - st
        cpos_ps = cpos_ps + ce * posf
        np_ps = np_ps + pos.astype(jnp.int32)
        neg = jnp.where(pos, 0.0, ce)
        neg = jnp.where(easy, 0.0, neg)
        neg_ref[_B + i, :, sl] = _pad_mask(c, neg)
    n_pos_o = jnp.sum(np_ps)
    kv_ref[pl.ds(_B + i, 1), :] = jnp.full((1, 128),
                                           _NEG_POS_RATIO * n_pos_o,
                                           jnp.int32)
    loc_o = jnp.sum(loc_ps)
    cpos_o = jnp.sum(cpos_ps)

    @pl.when(i == 0)
    def _init():
        for t in range(6):
            acc_ref[t] = 0.0

    parts = (loc_a, cpos_a, n_pos_a.astype(jnp.float32),
             loc_o, cpos_o, n_pos_o.astype(jnp.float32))
    for t, v in enumerate(parts):
        acc_ref[t] = acc_ref[t] + v

    @pl.when(i == _B - 1)
    def _fin():
        hard_a, hard_o = _batched_topk_sums(neg_ref, kv_ref)
        na = acc_ref[2]
        no = acc_ref[5]
        arm = (hard_a + acc_ref[1]) / na + _ALPHA * acc_ref[0] / (na * 4.0)
        odm = (hard_o + acc_ref[4]) / no + _ALPHA * acc_ref[3] / (no * 4.0)
        out_ref[0, 0] = arm + odm


def _prep(x, k):
    xp = jnp.pad(x, ((0, 0), (0, _PF - _P), (0, 0)))
    return xp.transpose(0, 2, 1).reshape(_B, k, _R, _C)


def kernel(arm_locs, arm_scores, odm_locs, odm_scores, boxes, labels,
           priors_cxcy):
    al = _prep(arm_locs, 4)
    asr = _prep(arm_scores, 2)
    ol = _prep(odm_locs, 4)
    osr = _prep(odm_scores, _NC)
    pad_pr = jnp.tile(jnp.array([[1e6, 1e6, 1.0, 1.0]], jnp.float32),
                      (_PF - _P, 1))
    pr = jnp.concatenate([priors_cxcy, pad_pr], axis=0).T.reshape(4, _R, _C)
    out = pl.pallas_call(
        _body,
        grid=(_B,),
        in_specs=[
            pl.BlockSpec((4, _R, _C), lambda i: (0, 0, 0)),
            pl.BlockSpec((1, _NOBJ, 4), lambda i: (i, 0, 0),
                         memory_space=pltpu.SMEM),
            pl.BlockSpec((1, 1, _NOBJ), lambda i: (i, 0, 0),
                         memory_space=pltpu.SMEM),
            pl.BlockSpec((1, 4, _R, _C), lambda i: (i, 0, 0, 0)),
            pl.BlockSpec((1, 2, _R, _C), lambda i: (i, 0, 0, 0)),
            pl.BlockSpec((1, 4, _R, _C), lambda i: (i, 0, 0, 0)),
            pl.BlockSpec((1, _NC, _R, _C), lambda i: (i, 0, 0, 0)),
        ],
        out_specs=pl.BlockSpec((1, 1), lambda i: (0, 0),
                               memory_space=pltpu.SMEM),
        out_shape=jax.ShapeDtypeStruct((1, 1), jnp.float32),
        scratch_shapes=[pltpu.SMEM((8,), jnp.float32),
                        pltpu.VMEM((_NROW, _R, _C), jnp.float32),
                        pltpu.VMEM((_NROW, 128), jnp.int32),
                        pltpu.VMEM((_NOBJ, _R, _C), jnp.float32),
                        pltpu.VMEM((_R, _C), jnp.float32),
                        pltpu.VMEM((_R, _C), jnp.int32),
                        pltpu.VMEM((4, _R, _C), jnp.float32)],
    )(pr, boxes, labels.astype(jnp.int32).reshape(_B, 1, _NOBJ),
      al, asr, ol, osr)
    return out[0, 0]


# speedup vs baseline: 1.0065x; 1.0065x over previous
import jax
import jax.numpy as jnp
from jax import lax
from jax.experimental import pallas as pl
from jax.experimental.pallas import tpu as pltpu

_B, _P, _NOBJ, _NC = 16, 16320, 12, 21
_R, _C = 8, 2048
_PF = _R * _C
_W = 128
_NCH = _C // _W
_THRESHOLD, _NEG_POS_RATIO, _THETA, _ALPHA = 0.5, 3, 0.01, 1.0
_NROW = 2 * _B


def _chunk_idx(c):
    r = lax.broadcasted_iota(jnp.int32, (_R, _W), 0)
    l = lax.broadcasted_iota(jnp.int32, (_R, _W), 1)
    return r * _C + (c * _W + l)


def _full_idx():
    r = lax.broadcasted_iota(jnp.int32, (_R, _C), 0)
    l = lax.broadcasted_iota(jnp.int32, (_R, _C), 1)
    return r * _C + l


def _box_scalars(boxes_ref, u):
    out = []
    for j in range(_NOBJ):
        bx1 = boxes_ref[u, j, 0]
        by1 = boxes_ref[u, j, 1]
        bx2 = boxes_ref[u, j, 2]
        by2 = boxes_ref[u, j, 3]
        out.append((bx1, by1, bx2, by2, (bx2 - bx1) * (by2 - by1)))
    return out

def _iou_stage_a(bx, anchors_fn, iou_scr, best_scr, obj_scr):
    for c in range(_NCH):
        sl = pl.ds(c * _W, _W)
        ax1, ay1, ax2, ay2 = anchors_fn(c)
        area_b = (ax2 - ax1) * (ay2 - ay1)
        best = None
        obj = None
        for j in range(_NOBJ):
            bx1, by1, bx2, by2, area_a = bx[j]
            w = jnp.maximum(jnp.minimum(bx2, ax2) - jnp.maximum(bx1, ax1),
                            0.0)
            h = jnp.maximum(jnp.minimum(by2, ay2) - jnp.maximum(by1, ay1),
                            0.0)
            inter = w * h
            iou = inter / (area_a + area_b - inter)
            iou_scr[j, :, sl] = iou
            if j == 0:
                best = iou
                obj = jnp.zeros((_R, _W), jnp.int32)
            else:
                obj = jnp.where(iou > best, j, obj)
                best = jnp.maximum(best, iou)
        best_scr[:, sl] = best
        obj_scr[:, sl] = obj


def _rank_stage_b(iou_scr):
    fidx = _full_idx()
    pfs, svs = [], []
    rank = jnp.zeros((1, 1), jnp.int32) - 1
    for j in range(_NOBJ):
        ioj = iou_scr[j]
        mx = jnp.max(ioj, axis=(0, 1), keepdims=True)
        mxb = jnp.broadcast_to(mx, (_R, _C))
        pf = jnp.min(jnp.where(ioj == mxb, fidx, _PF), axis=(0, 1),
                     keepdims=True)
        valid = mx > 0.0
        rank = rank + valid.astype(jnp.int32)
        sv = jnp.where(valid, rank + 1, 0)
        svs.append(jnp.broadcast_to(sv, (_R, _W)))
        pfs.append(jnp.broadcast_to(pf, (_R, _W)))
    return pfs, svs


def _finish_chunk(c, bx, lbl, best_scr, obj_scr, pfs, svs, cxcy):
    sl = pl.ds(c * _W, _W)
    fidx = _chunk_idx(c)
    best = best_scr[:, sl]
    obj = obj_scr[:, sl]
    force = jnp.zeros((_R, _W), jnp.int32)
    for j in range(_NOBJ):
        force = jnp.maximum(force, jnp.where(fidx == pfs[j], svs[j], 0))
    forced = force > 0
    best = jnp.where(forced, 1.0, best)
    obj = jnp.where(forced, force - 1, obj)
    lab = jnp.zeros((_R, _W), jnp.int32)
    gx1 = jnp.zeros((_R, _W), jnp.float32)
    gy1 = jnp.zeros((_R, _W), jnp.float32)
    gx2 = jnp.zeros((_R, _W), jnp.float32)
    gy2 = jnp.zeros((_R, _W), jnp.float32)
    for j in range(_NOBJ):
        sel = obj == j
        lab = jnp.where(sel, lbl[j], lab)
        gx1 = jnp.where(sel, bx[j][0], gx1)
        gy1 = jnp.where(sel, bx[j][1], gy1)
        gx2 = jnp.where(sel, bx[j][2], gx2)
        gy2 = jnp.where(sel, bx[j][3], gy2)
    lab = jnp.where(best < _THRESHOLD, 0, lab)
    pcx, pcy, pw, ph = cxcy
    cx = (gx1 + gx2) / 2.0
    cy = (gy1 + gy2) / 2.0
    w = gx2 - gx1
    h = gy2 - gy1
    t0 = (cx - pcx) / (pw / 10.0)
    t1 = (cy - pcy) / (ph / 10.0)
    t2 = jnp.log(w / pw) * 5.0
    t3 = jnp.log(h / ph) * 5.0
    return lab, (t0, t1, t2, t3)


def _pad_mask(c, x):
    if c == _NCH - 1:
        return jnp.where(_chunk_idx(c) < _P, x, 0.0)
    return x


def _batched_topk_sums(neg_ref, kv_ref):
    kcol = kv_ref[:, 0:1]

    def count_ge(mid):
        cnt = jnp.zeros((_NROW, 128), jnp.int32)
        m3 = mid[:, None, :]
        for c in range(_NCH):
            blk = lax.bitcast_convert_type(
                neg_ref[:, :, pl.ds(c * _W, _W)], jnp.int32)
            cnt = cnt + jnp.sum((blk >= m3).astype(jnp.int32), axis=1)
        return jnp.sum(cnt, axis=1, keepdims=True)

    def body(_, lohi):
        lo, hi = lohi
        mid = lo + (hi - lo + 1) // 2
        ok = count_ge(mid) >= kcol
        return jnp.where(ok, mid, lo), jnp.where(ok, hi, mid - 1)

    lo, _ = lax.fori_loop(
        0, 31, body,
        (jnp.zeros((_NROW, 1), jnp.int32),
         jnp.full((_NROW, 1), 0x7F800000, jnp.int32)))
    tf = lax.bitcast_convert_type(lo, jnp.float32)
    srel = jnp.zeros((_NROW, 128), jnp.float32)
    t3 = tf[:, None, :]
    for c in range(_NCH):
        blk = neg_ref[:, :, pl.ds(c * _W, _W)]
        srel = srel + jnp.sum(jnp.maximum(blk - t3, 0.0), axis=1)
    s = jnp.sum(srel, axis=1, keepdims=True)
    kf = kcol.astype(jnp.float32)
    hard = jnp.where(kcol > 0, kf * tf + s, 0.0)
    return jnp.sum(hard[:_B, 0]), jnp.sum(hard[_B:, 0])


def _one_image(u, img, pr_ref, boxes_ref, labels_ref, al_ref, as_ref,
               ol_ref, os_ref, neg_ref, kv_ref, iou_scr, best_scr, obj_scr,
               dec_scr):
    bx = _box_scalars(boxes_ref, u)
    lbl = [labels_ref[u, 0, j] for j in range(_NOBJ)]

    def arm_anchors(c):
        sl = pl.ds(c * _W, _W)
        pcx, pcy, pw, ph = (pr_ref[0, :, sl], pr_ref[1, :, sl],
                            pr_ref[2, :, sl], pr_ref[3, :, sl])
        return (pcx - pw / 2.0, pcy - ph / 2.0,
                pcx + pw / 2.0, pcy + ph / 2.0)

    _iou_stage_a(bx, arm_anchors, iou_scr, best_scr, obj_scr)
    pfs, svs = _rank_stage_b(iou_scr)
    loc_ps = jnp.zeros((_R, _W), jnp.float32)
    cpos_ps = jnp.zeros((_R, _W), jnp.float32)
    np_ps = jnp.zeros((_R, _W), jnp.int32)
    for c in range(_NCH):
        sl = pl.ds(c * _W, _W)
        cxcy = (pr_ref[0, :, sl], pr_ref[1, :, sl],
                pr_ref[2, :, sl], pr_ref[3, :, sl])
        lab, ta = _finish_chunk(c, bx, lbl, best_scr, obj_scr, pfs, svs,
                                cxcy)
        pos = lab > 0
        posf = pos.astype(jnp.float32)
        for k in range(4):
            d = jnp.abs(al_ref[u, k, :, sl] - ta[k])
            loc_ps = loc_ps + jnp.where(d < 1.0, 0.5 * d * d,
                                        d - 0.5) * posf
        s0 = as_ref[u, 0, :, sl]
        s1 = as_ref[u, 1, :, sl]
        m2 = jnp.maximum(s0, s1)
        lse = m2 + jnp.log(jnp.exp(s0 - m2) + jnp.exp(s1 - m2))
        ce = lse - jnp.where(pos, s1, s0)
        cpos_ps = cpos_ps + ce * posf
        np_ps = np_ps + pos.astype(jnp.int32)
        neg_ref[img, :, sl] = _pad_mask(c, jnp.where(pos, 0.0, ce))
    n_pos_a = jnp.sum(np_ps)
    kv_ref[pl.ds(img, 1), :] = jnp.full((1, 128), _NEG_POS_RATIO * n_pos_a,
                                        jnp.int32)
    loc_a = jnp.sum(loc_ps)
    cpos_a = jnp.sum(cpos_ps)

    def odm_anchors(c):
        sl = pl.ds(c * _W, _W)
        pcx, pcy, pw, ph = (pr_ref[0, :, sl], pr_ref[1, :, sl],
                            pr_ref[2, :, sl], pr_ref[3, :, sl])
        a0, a1, a2, a3 = (al_ref[u, 0, :, sl], al_ref[u, 1, :, sl],
                          al_ref[u, 2, :, sl], al_ref[u, 3, :, sl])
        dcx = a0 * pw / 10.0 + pcx
        dcy = a1 * ph / 10.0 + pcy
        dw = jnp.exp(a2 / 5.0) * pw
        dh = jnp.exp(a3 / 5.0) * ph
        dx1 = dcx - dw / 2.0
        dy1 = dcy - dh / 2.0
        dx2 = dcx + dw / 2.0
        dy2 = dcy + dh / 2.0
        dec_scr[0, :, sl] = (dx1 + dx2) / 2.0
        dec_scr[1, :, sl] = (dy1 + dy2) / 2.0
        dec_scr[2, :, sl] = dx2 - dx1
        dec_scr[3, :, sl] = dy2 - dy1
        return dx1, dy1, dx2, dy2

    _iou_stage_a(bx, odm_anchors, iou_scr, best_scr, obj_scr)
    pfs, svs = _rank_stage_b(iou_scr)
    loc_ps = jnp.zeros((_R, _W), jnp.float32)
    cpos_ps = jnp.zeros((_R, _W), jnp.float32)
    np_ps = jnp.zeros((_R, _W), jnp.int32)
    for c in range(_NCH):
        sl = pl.ds(c * _W, _W)
        cxcy = (dec_scr[0, :, sl], dec_scr[1, :, sl],
                dec_scr[2, :, sl], dec_scr[3, :, sl])
        lab, to = _finish_chunk(c, bx, lbl, best_scr, obj_scr, pfs, svs,
                                cxcy)
        s0 = as_ref[u, 0, :, sl]
        s1 = as_ref[u, 1, :, sl]
        em = jnp.maximum(s0, s1)
        e0 = jnp.exp(s0 - em)
        e1 = jnp.exp(s1 - em)
        easy = e1 / (e0 + e1) < _THETA
        pos = jnp.logical_and(lab > 0, jnp.logical_not(easy))
        posf = pos.astype(jnp.float32)
        for k in range(4):
            d = jnp.abs(ol_ref[u, k, :, sl] - to[k])
            loc_ps = loc_ps + jnp.where(d < 1.0, 0.5 * d * d,
                                        d - 0.5) * posf
        sc = [os_ref[u, cc, :, sl] for cc in range(_NC)]
        mo = sc[0]
        for cc in range(1, _NC):
            mo = jnp.maximum(mo, sc[cc])
        se = jnp.zeros((_R, _W), jnp.float32)
        st = jnp.zeros((_R, _W), jnp.float32)
        for cc in range(_NC):
            se = se + jnp.exp(sc[cc] - mo)
            st = st + jnp.where(lab == cc, sc[cc], 0.0)
        ce = (mo + jnp.log(se)) - st
        cpos_ps = cpos_ps + ce * posf
        np_ps = np_ps + pos.astype(jnp.int32)
        neg = jnp.where(pos, 0.0, ce)
        neg = jnp.where(easy, 0.0, neg)
        neg_ref[_B + img, :, sl] = _pad_mask(c, neg)
    n_pos_o = jnp.sum(np_ps)
    kv_ref[pl.ds(_B + img, 1), :] = jnp.full((1, 128),
                                             _NEG_POS_RATIO * n_pos_o,
                                             jnp.int32)
    loc_o = jnp.sum(loc_ps)
    cpos_o = jnp.sum(cpos_ps)
    return (loc_a, cpos_a, n_pos_a.astype(jnp.float32),
            loc_o, cpos_o, n_pos_o.astype(jnp.float32))


def _body(pr_ref, boxes_ref, labels_ref, al_ref, as_ref, ol_ref, os_ref,
          out_ref, acc_ref, neg_ref, kv_ref, iou_scr, best_scr, obj_scr,
          dec_scr):
    i = pl.program_id(0)

    @pl.when(i == 0)
    def _init():
        for t in range(6):
            acc_ref[t] = 0.0

    for u in range(2):
        parts = _one_image(u, 2 * i + u, pr_ref, boxes_ref, labels_ref,
                           al_ref, as_ref, ol_ref, os_ref, neg_ref, kv_ref,
                           iou_scr.at[u], best_scr.at[u], obj_scr.at[u],
                           dec_scr.at[u])
        for t, v in enumerate(parts):
            acc_ref[t] = acc_ref[t] + v

    @pl.when(i == _B // 2 - 1)
    def _fin():
        hard_a, hard_o = _batched_topk_sums(neg_ref, kv_ref)
        na = acc_ref[2]
        no = acc_ref[5]
        arm = (hard_a + acc_ref[1]) / na + _ALPHA * acc_ref[0] / (na * 4.0)
        odm = (hard_o + acc_ref[4]) / no + _ALPHA * acc_ref[3] / (no * 4.0)
        out_ref[0, 0] = arm + odm


def _prep(x, k):
    xp = jnp.pad(x, ((0, 0), (0, _PF - _P), (0, 0)))
    return xp.transpose(0, 2, 1).reshape(_B, k, _R, _C)


def kernel(arm_locs, arm_scores, odm_locs, odm_scores, boxes, labels,
           priors_cxcy):
    al = _prep(arm_locs, 4)
    asr = _prep(arm_scores, 2)
    ol = _prep(odm_locs, 4)
    osr = _prep(odm_scores, _NC)
    pad_pr = jnp.tile(jnp.array([[1e6, 1e6, 1.0, 1.0]], jnp.float32),
                      (_PF - _P, 1))
    pr = jnp.concatenate([priors_cxcy, pad_pr], axis=0).T.reshape(4, _R, _C)
    out = pl.pallas_call(
        _body,
        grid=(_B // 2,),
        in_specs=[
            pl.BlockSpec((4, _R, _C), lambda i: (0, 0, 0)),
            pl.BlockSpec((2, _NOBJ, 4), lambda i: (i, 0, 0),
                         memory_space=pltpu.SMEM),
            pl.BlockSpec((2, 1, _NOBJ), lambda i: (i, 0, 0),
                         memory_space=pltpu.SMEM),
            pl.BlockSpec((2, 4, _R, _C), lambda i: (i, 0, 0, 0)),
            pl.BlockSpec((2, 2, _R, _C), lambda i: (i, 0, 0, 0)),
            pl.BlockSpec((2, 4, _R, _C), lambda i: (i, 0, 0, 0)),
            pl.BlockSpec((2, _NC, _R, _C), lambda i: (i, 0, 0, 0)),
        ],
        out_specs=pl.BlockSpec((1, 1), lambda i: (0, 0),
                               memory_space=pltpu.SMEM),
        out_shape=jax.ShapeDtypeStruct((1, 1), jnp.float32),
        scratch_shapes=[pltpu.SMEM((8,), jnp.float32),
                        pltpu.VMEM((_NROW, _R, _C), jnp.float32),
                        pltpu.VMEM((_NROW, 128), jnp.int32),
                        pltpu.VMEM((2, _NOBJ, _R, _C), jnp.float32),
                        pltpu.VMEM((2, _R, _C), jnp.float32),
                        pltpu.VMEM((2, _R, _C), jnp.int32),
                        pltpu.VMEM((2, 4, _R, _C), jnp.float32)],
    )(pr, boxes, labels.astype(jnp.int32).reshape(_B, 1, _NOBJ),
      al, asr, ol, osr)
    return out[0, 0]


# speedup vs baseline: 1.0412x; 1.0344x over previous
import jax
import jax.numpy as jnp
from jax import lax
from jax.experimental import pallas as pl
from jax.experimental.pallas import tpu as pltpu

_B, _P, _NOBJ, _NC = 16, 16320, 12, 21
_R, _C = 8, 2040
_THRESHOLD, _NEG_POS_RATIO, _THETA, _ALPHA = 0.5, 3, 0.01, 1.0


def _flat_idx():
    r = lax.broadcasted_iota(jnp.int32, (_R, _C), 0)
    c = lax.broadcasted_iota(jnp.int32, (_R, _C), 1)
    return r * _C + c


def _match(boxes_ref, labels_ref, ax1, ay1, ax2, ay2, pcx, pcy, pw, ph):
    area_b = (ax2 - ax1) * (ay2 - ay1)
    fidx = _flat_idx()
    best = None
    obj = None
    mxs, pfs = [], []
    for j in range(_NOBJ):
        bx1 = boxes_ref[0, j, 0]
        by1 = boxes_ref[0, j, 1]
        bx2 = boxes_ref[0, j, 2]
        by2 = boxes_ref[0, j, 3]
        w = jnp.maximum(jnp.minimum(bx2, ax2) - jnp.maximum(bx1, ax1), 0.0)
        h = jnp.maximum(jnp.minimum(by2, ay2) - jnp.maximum(by1, ay1), 0.0)
        inter = w * h
        area_a = (bx2 - bx1) * (by2 - by1)
        iou = inter / (area_a + area_b - inter)
        mx = jnp.max(iou, axis=(0, 1), keepdims=True)
        pfs.append(jnp.min(jnp.where(iou == mx, fidx, _P), axis=(0, 1),
                           keepdims=True))
        mxs.append(mx)
        if j == 0:
            best = iou
            obj = jnp.zeros((_R, _C), jnp.int32)
        else:
            obj = jnp.where(iou > best, j, obj)
            best = jnp.maximum(best, iou)
    rank = jnp.zeros((1, 1), jnp.int32) - 1
    force = jnp.zeros((_R, _C), jnp.int32)
    for j in range(_NOBJ):
        valid = mxs[j] > 0.0
        rank = rank + valid.astype(jnp.int32)
        sv = jnp.where(valid, rank + 1, 0)
        force = jnp.maximum(force, jnp.where(fidx == pfs[j], sv, 0))
    forced = force > 0
    best = jnp.where(forced, 1.0, best)
    obj = jnp.where(forced, force - 1, obj)
    lab = jnp.zeros((_R, _C), jnp.int32)
    gx1 = jnp.zeros((_R, _C), jnp.float32)
    gy1 = jnp.zeros((_R, _C), jnp.float32)
    gx2 = jnp.zeros((_R, _C), jnp.float32)
    gy2 = jnp.zeros((_R, _C), jnp.float32)
    for j in range(_NOBJ):
        sel = obj == j
        lab = jnp.where(sel, labels_ref[0, 0, j], lab)
        gx1 = jnp.where(sel, boxes_ref[0, j, 0], gx1)
        gy1 = jnp.where(sel, boxes_ref[0, j, 1], gy1)
        gx2 = jnp.where(sel, boxes_ref[0, j, 2], gx2)
        gy2 = jnp.where(sel, boxes_ref[0, j, 3], gy2)
    lab = jnp.where(best < _THRESHOLD, 0, lab)
    cx = (gx1 + gx2) / 2.0
    cy = (gy1 + gy2) / 2.0
    w = gx2 - gx1
    h = gy2 - gy1
    t0 = (cx - pcx) / (pw / 10.0)
    t1 = (cy - pcy) / (ph / 10.0)
    t2 = jnp.log(w / pw) * 5.0
    t3 = jnp.log(h / ph) * 5.0
    return lab, (t0, t1, t2, t3)


_C2 = 2048
_NROW = 2 * _B


def _batched_topk_sums(neg_ref, kv_ref):
    kcol = kv_ref[:, 0:1]

    def count_ge(mid):
        cnt = jnp.zeros((_NROW, 128), jnp.int32)
        m3 = mid[:, None, :]
        for c in range(_C2 // 128):
            blk = lax.bitcast_convert_type(
                neg_ref[:, :, pl.ds(c * 128, 128)], jnp.int32)
            cnt = cnt + jnp.sum((blk >= m3).astype(jnp.int32), axis=1)
        return jnp.sum(cnt, axis=1, keepdims=True)

    def body(_, lohi):
        lo, hi = lohi
        mid = lo + (hi - lo + 1) // 2
        ok = count_ge(mid) >= kcol
        return jnp.where(ok, mid, lo), jnp.where(ok, hi, mid - 1)

    lo, _ = lax.fori_loop(
        0, 31, body,
        (jnp.zeros((_NROW, 1), jnp.int32),
         jnp.full((_NROW, 1), 0x7F800000, jnp.int32)))
    tf = lax.bitcast_convert_type(lo, jnp.float32)
    srel = jnp.zeros((_NROW, 128), jnp.float32)
    t3 = tf[:, None, :]
    for c in range(_C2 // 128):
        blk = neg_ref[:, :, pl.ds(c * 128, 128)]
        srel = srel + jnp.sum(jnp.maximum(blk - t3, 0.0), axis=1)
    s = jnp.sum(srel, axis=1, keepdims=True)
    kf = kcol.astype(jnp.float32)
    hard = jnp.where(kcol > 0, kf * tf + s, 0.0)
    return jnp.sum(hard[:_B, 0]), jnp.sum(hard[_B:, 0])


def _loc_loss_sum(pred, tgt, posf):
    acc = jnp.float32(0.0)
    for c in range(4):
        d = jnp.abs(pred[c] - tgt[c])
        acc = acc + jnp.sum(jnp.where(d < 1.0, 0.5 * d * d, d - 0.5) * posf)
    return acc


def _body(pr_ref, boxes_ref, labels_ref, al_ref, as_ref, ol_ref, os_ref,
          out_ref, acc_ref, neg_ref, kv_ref):
    i = pl.program_id(0)

    @pl.when(i == 0)
    def _zero():
        neg_ref[...] = jnp.zeros((_NROW, _R, _C2), jnp.float32)

    pcx = pr_ref[0]
    pcy = pr_ref[1]
    pw = pr_ref[2]
    ph = pr_ref[3]
    px1 = pcx - pw / 2.0
    py1 = pcy - ph / 2.0
    px2 = pcx + pw / 2.0
    py2 = pcy + ph / 2.0

    lab_a, ta = _match(boxes_ref, labels_ref, px1, py1, px2, py2,
                       pcx, pcy, pw, ph)
    pos_a = lab_a > 0
    posf_a = pos_a.astype(jnp.float32)
    n_pos_a = jnp.sum(posf_a)
    al = [al_ref[0, c] for c in range(4)]
    loc_a = _loc_loss_sum(al, ta, posf_a)
    s0 = as_ref[0, 0]
    s1 = as_ref[0, 1]
    m2 = jnp.maximum(s0, s1)
    lse2 = m2 + jnp.log(jnp.exp(s0 - m2) + jnp.exp(s1 - m2))
    ce_a = lse2 - jnp.where(pos_a, s1, s0)
    cpos_a = jnp.sum(ce_a * posf_a)
    neg_a = jnp.where(pos_a, 0.0, ce_a)
    k_a = _NEG_POS_RATIO * jnp.sum(pos_a.astype(jnp.int32))
    neg_ref[i, :, pl.ds(0, _C)] = neg_a
    kv_ref[pl.ds(i, 1), :] = jnp.full((1, 128), k_a, jnp.int32)

    a0 = al[0]
    a1 = al[1]
    a2 = al[2]
    a3 = al[3]
    dcx = a0 * pw / 10.0 + pcx
    dcy = a1 * ph / 10.0 + pcy
    dw = jnp.exp(a2 / 5.0) * pw
    dh = jnp.exp(a3 / 5.0) * ph
    dx1 = dcx - dw / 2.0
    dy1 = dcy - dh / 2.0
    dx2 = dcx + dw / 2.0
    dy2 = dcy + dh / 2.0
    ocx = (dx2 + dx1) / 2.0
    ocy = (dy2 + dy1) / 2.0
    ow = dx2 - dx1
    oh = dy2 - dy1
    lab_o, to = _match(boxes_ref, labels_ref, dx1, dy1, dx2, dy2,
                       ocx, ocy, ow, oh)
    em = jnp.maximum(s0, s1)
    e0 = jnp.exp(s0 - em)
    e1 = jnp.exp(s1 - em)
    easy = e1 / (e0 + e1) < _THETA
    pos_o = jnp.logical_and(lab_o > 0, jnp.logical_not(easy))
    posf_o = pos_o.astype(jnp.float32)
    n_pos_o = jnp.sum(posf_o)
    ol = [ol_ref[0, c] for c in range(4)]
    loc_o = _loc_loss_sum(ol, to, posf_o)
    sc = [os_ref[0, c] for c in range(_NC)]
    mo = sc[0]
    for c in range(1, _NC):
        mo = jnp.maximum(mo, sc[c])
    se = jnp.zeros((_R, _C), jnp.float32)
    st = jnp.zeros((_R, _C), jnp.float32)
    for c in range(_NC):
        se = se + jnp.exp(sc[c] - mo)
        st = st + jnp.where(lab_o == c, sc[c], 0.0)
    ce_o = (mo + jnp.log(se)) - st
    cpos_o = jnp.sum(ce_o * posf_o)
    neg_o = jnp.where(pos_o, 0.0, ce_o)
    neg_o = jnp.where(easy, 0.0, neg_o)
    k_o = _NEG_POS_RATIO * jnp.sum(pos_o.astype(jnp.int32))
    neg_ref[_B + i, :, pl.ds(0, _C)] = neg_o
    kv_ref[pl.ds(_B + i, 1), :] = jnp.full((1, 128), k_o, jnp.int32)

    @pl.when(i == 0)
    def _init():
        for t in range(6):
            acc_ref[t] = 0.0

    parts = (loc_a, cpos_a, n_pos_a, loc_o, cpos_o, n_pos_o)
    for t, v in enumerate(parts):
        acc_ref[t] = acc_ref[t] + v

    @pl.when(i == _B - 1)
    def _fin():
        hard_a, hard_o = _batched_topk_sums(neg_ref, kv_ref)
        na = acc_ref[2]
        no = acc_ref[5]
        arm = (hard_a + acc_ref[1]) / na + _ALPHA * acc_ref[0] / (na * 4.0)
        odm = (hard_o + acc_ref[4]) / no + _ALPHA * acc_ref[3] / (no * 4.0)
        out_ref[0, 0] = arm + odm


def kernel(arm_locs, arm_scores, odm_locs, odm_scores, boxes, labels,
           priors_cxcy):
    al = arm_locs.transpose(0, 2, 1).reshape(_B, 4, _R, _C)
    asr = arm_scores.transpose(0, 2, 1).reshape(_B, 2, _R, _C)
    ol = odm_locs.transpose(0, 2, 1).reshape(_B, 4, _R, _C)
    osr = odm_scores.transpose(0, 2, 1).reshape(_B, _NC, _R, _C)
    pr = priors_cxcy.T.reshape(4, _R, _C)
    out = pl.pallas_call(
        _body,
        grid=(_B,),
        in_specs=[
            pl.BlockSpec((4, _R, _C), lambda i: (0, 0, 0)),
            pl.BlockSpec((1, _NOBJ, 4), lambda i: (i, 0, 0),
                         memory_space=pltpu.SMEM),
            pl.BlockSpec((1, 1, _NOBJ), lambda i: (i, 0, 0),
                         memory_space=pltpu.SMEM),
            pl.BlockSpec((1, 4, _R, _C), lambda i: (i, 0, 0, 0)),
            pl.BlockSpec((1, 2, _R, _C), lambda i: (i, 0, 0, 0)),
            pl.BlockSpec((1, 4, _R, _C), lambda i: (i, 0, 0, 0)),
            pl.BlockSpec((1, _NC, _R, _C), lambda i: (i, 0, 0, 0)),
        ],
        out_specs=pl.BlockSpec((1, 1), lambda i: (0, 0),
                               memory_space=pltpu.SMEM),
        out_shape=jax.ShapeDtypeStruct((1, 1), jnp.float32),
        scratch_shapes=[pltpu.SMEM((8,), jnp.float32),
                        pltpu.VMEM((_NROW, _R, _C2), jnp.float32),
                        pltpu.VMEM((_NROW, 128), jnp.int32)],
    )(pr, boxes, labels.astype(jnp.int32).reshape(_B, 1, _NOBJ),
      al, asr, ol, osr)
    return out[0, 0]
